# Initial kernel scaffold; baseline (speedup 1.0000x reference)
import jax
import jax.numpy as jnp
from jax import lax
from jax.experimental import pallas as pl
from jax.experimental.pallas import tpu as pltpu
from jax.experimental.pallas import tpu_sc as plsc

D = 16
L = 200
B = 16384
NC, NS = 2, 16
NW = NC * NS
BAGS_PER_W = B // NW
G = 16
NG = BAGS_PER_W // G
IDX_PER_G = G * L
BLK = 128
NBLK = IDX_PER_G // BLK


def _fire_group(text_hbm, table_hbm, idx_v, rows_v, sem, base):
    pltpu.sync_copy(text_hbm.at[pl.ds(base, IDX_PER_G)], idx_v)
    for j in range(NBLK):
        pltpu.async_copy(
            table_hbm.at[idx_v.at[pl.ds(j * BLK, BLK)]],
            rows_v.at[pl.ds(j * BLK, BLK)],
            sem,
        )


def _drain_group(table_hbm, rows_v, sem):
    pltpu.make_async_copy(table_hbm.at[pl.ds(0, IDX_PER_G)], rows_v, sem).wait()


def _process_group(rows_v, wt_rows, b_vec, out_v, out_hbm, obase):
    def bag_body(bag, carry):
        rbase = bag * L

        def rbody(i, accs):
            a0, a1, a2, a3 = accs
            r = rbase + i * 8
            a0 = a0 + (rows_v[r, :] + rows_v[r + 4, :])
            a1 = a1 + (rows_v[r + 1, :] + rows_v[r + 5, :])
            a2 = a2 + (rows_v[r + 2, :] + rows_v[r + 6, :])
            a3 = a3 + (rows_v[r + 3, :] + rows_v[r + 7, :])
            return a0, a1, a2, a3

        z = jnp.zeros((D,), jnp.float32)
        a0, a1, a2, a3 = lax.fori_loop(0, L // 8, rbody, (z, z, z, z))
        s = (a0 + a1) + (a2 + a3)

        parts = [b_vec, z, z, z]
        for k in range(D):
            parts[k % 4] = parts[k % 4] + s[k] * wt_rows[k]
        out_v[bag, :] = (parts[0] + parts[1]) + (parts[2] + parts[3])
        return carry

    lax.fori_loop(0, G, bag_body, 0)
    pltpu.sync_copy(out_v, out_hbm.at[pl.ds(obase, G), :])


def _body(text_hbm, table_hbm, wt_hbm, bias_hbm, out_hbm,
          idx0, idx1, rows0, rows1, wt_v, b_v, out_v, sem0, sem1):
    wid = lax.axis_index("s") * NC + lax.axis_index("c")
    tbase = wid * BAGS_PER_W * L
    obase = wid * BAGS_PER_W

    pltpu.sync_copy(wt_hbm, wt_v)
    pltpu.sync_copy(bias_hbm, b_v)
    wt_rows = [wt_v[k, :] for k in range(D)]
    b_vec = b_v[:]

    _fire_group(text_hbm, table_hbm, idx0, rows0, sem0, tbase)

    def outer(g2, carry):
        gA = g2 * 2
        gB = gA + 1
        _fire_group(text_hbm, table_hbm, idx1, rows1, sem1,
                    tbase + gB * IDX_PER_G)
        _drain_group(table_hbm, rows0, sem0)
        _process_group(rows0, wt_rows, b_vec, out_v, out_hbm, obase + gA * G)

        @pl.when(g2 < NG // 2 - 1)
        def _():
            _fire_group(text_hbm, table_hbm, idx0, rows0, sem0,
                        tbase + (gA + 2) * IDX_PER_G)

        _drain_group(table_hbm, rows1, sem1)
        _process_group(rows1, wt_rows, b_vec, out_v, out_hbm, obase + gB * G)
        return carry

    lax.fori_loop(0, NG // 2, outer, 0)


@jax.jit
def kernel(text, table, W, b):
    text_flat = text.reshape(-1).astype(jnp.int32)
    wt = (W.T / jnp.float32(L)).astype(jnp.float32)
    run = pl.kernel(
        _body,
        out_type=jax.ShapeDtypeStruct((B, D), jnp.float32),
        mesh=plsc.VectorSubcoreMesh(core_axis_name="c", subcore_axis_name="s"),
        scratch_types=[
            pltpu.VMEM((IDX_PER_G,), jnp.int32),
            pltpu.VMEM((IDX_PER_G,), jnp.int32),
            pltpu.VMEM((IDX_PER_G, D), jnp.float32),
            pltpu.VMEM((IDX_PER_G, D), jnp.float32),
            pltpu.VMEM((D, D), jnp.float32),
            pltpu.VMEM((D,), jnp.float32),
            pltpu.VMEM((G, D), jnp.float32),
            pltpu.SemaphoreType.DMA,
            pltpu.SemaphoreType.DMA,
        ],
    )
    return run(text_flat, table, wt, b.astype(jnp.float32))


# speedup vs baseline: 9.5736x; 9.5736x over previous
import jax
import jax.numpy as jnp
from jax import lax
from jax.experimental import pallas as pl
from jax.experimental.pallas import tpu as pltpu
from jax.experimental.pallas import tpu_sc as plsc

D = 16
L = 200
B = 16384
NC, NS = 2, 16
NW = NC * NS
BAGS_PER_W = B // NW
G = 16
NG = BAGS_PER_W // G
IDX_PER_G = G * L
BLK = 128
NBLK = IDX_PER_G // BLK


def _fire_group(text_hbm, table_hbm, idx_v, rows_v, sem, base):
    pltpu.sync_copy(text_hbm.at[pl.ds(base, IDX_PER_G)], idx_v)
    for j in range(NBLK):
        pltpu.async_copy(
            table_hbm.at[idx_v.at[pl.ds(j * BLK, BLK)]],
            rows_v.at[pl.ds(j * BLK, BLK)],
            sem,
        )


def _drain_group(table_hbm, rows_v, sem):
    pltpu.make_async_copy(table_hbm.at[pl.ds(0, IDX_PER_G)], rows_v, sem).wait()


def _process_group(rows_v, wt_rows, b_vec, out_v, out_hbm, obase):
    def bag_body(bag, carry):
        rbase = bag * L

        def rbody(i, accs):
            a0, a1, a2, a3 = accs
            r = rbase + i * 8
            a0 = a0 + (rows_v[r, :] + rows_v[r + 4, :])
            a1 = a1 + (rows_v[r + 1, :] + rows_v[r + 5, :])
            a2 = a2 + (rows_v[r + 2, :] + rows_v[r + 6, :])
            a3 = a3 + (rows_v[r + 3, :] + rows_v[r + 7, :])
            return a0, a1, a2, a3

        z = jnp.zeros((D,), jnp.float32)
        a0, a1, a2, a3 = lax.fori_loop(0, L // 8, rbody, (z, z, z, z))
        s = (a0 + a1) + (a2 + a3)

        parts = [b_vec, z, z, z]
        for k in range(D):
            parts[k % 4] = parts[k % 4] + s[k] * wt_rows[k]
        out_v[bag, :] = (parts[0] + parts[1]) + (parts[2] + parts[3])
        return carry

    lax.fori_loop(0, G, bag_body, 0)
    pltpu.sync_copy(out_v, out_hbm.at[pl.ds(obase, G), :])


def _body(text_hbm, table_hbm, wt_hbm, bias_hbm, out_hbm,
          idx0, idx1, rows0, rows1, wt_v, b_v, out_v, sem0, sem1):
    wid = lax.axis_index("s") * NC + lax.axis_index("c")
    tbase = wid * BAGS_PER_W * L
    obase = wid * BAGS_PER_W

    pltpu.sync_copy(wt_hbm, wt_v)
    pltpu.sync_copy(bias_hbm, b_v)
    wt_rows = [wt_v[k, :] for k in range(D)]
    b_vec = b_v[:]

    _fire_group(text_hbm, table_hbm, idx0, rows0, sem0, tbase)

    def outer(g2, carry):
        gA = g2 * 2
        gB = gA + 1
        _fire_group(text_hbm, table_hbm, idx1, rows1, sem1,
                    tbase + gB * IDX_PER_G)
        _drain_group(table_hbm, rows0, sem0)
        _process_group(rows0, wt_rows, b_vec, out_v, out_hbm, obase + gA * G)

        @pl.when(g2 < NG // 2 - 1)
        def _():
            _fire_group(text_hbm, table_hbm, idx0, rows0, sem0,
                        tbase + (gA + 2) * IDX_PER_G)

        _drain_group(table_hbm, rows1, sem1)
        _process_group(rows1, wt_rows, b_vec, out_v, out_hbm, obase + gB * G)
        return carry

    lax.fori_loop(0, NG // 2, outer, 0)


@jax.jit
def kernel(text, table, W, b):
    text_flat = text.reshape(-1).astype(jnp.int32)
    wt = (W.T / jnp.float32(L)).astype(jnp.float32)
    run = pl.kernel(
        _body,
        out_type=jax.ShapeDtypeStruct((B, D), jnp.float32),
        mesh=plsc.VectorSubcoreMesh(core_axis_name="c", subcore_axis_name="s"),
        compiler_params=pltpu.CompilerParams(use_tc_tiling_on_sc=False),
        scratch_types=[
            pltpu.VMEM((IDX_PER_G,), jnp.int32),
            pltpu.VMEM((IDX_PER_G,), jnp.int32),
            pltpu.VMEM((IDX_PER_G, D), jnp.float32),
            pltpu.VMEM((IDX_PER_G, D), jnp.float32),
            pltpu.VMEM((D, D), jnp.float32),
            pltpu.VMEM((D,), jnp.float32),
            pltpu.VMEM((G, D), jnp.float32),
            pltpu.SemaphoreType.DMA,
            pltpu.SemaphoreType.DMA,
        ],
    )
    return run(text_flat, table, wt, b.astype(jnp.float32))


# speedup vs baseline: 12.0192x; 1.2555x over previous
import jax
import jax.numpy as jnp
from jax import lax
from jax.experimental import pallas as pl
from jax.experimental.pallas import tpu as pltpu
from jax.experimental.pallas import tpu_sc as plsc

D = 16
L = 200
B = 16384
NC, NS = 2, 16
NW = NC * NS
BAGS_PER_W = B // NW
G = 16
NG = BAGS_PER_W // G
IDX_PER_G = G * L
BLK = 128
NBLK = IDX_PER_G // BLK


def _fire_group(text_hbm, table_hbm, idx_v, rows_v, sem, base):
    pltpu.sync_copy(text_hbm.at[pl.ds(base, IDX_PER_G)], idx_v)
    for j in range(NBLK):
        pltpu.async_copy(
            table_hbm.at[idx_v.at[pl.ds(j * BLK, BLK)]],
            rows_v.at[pl.ds(j * BLK, BLK)],
            sem,
        )


def _drain_group(table_hbm, rows_v, sem):
    pltpu.make_async_copy(table_hbm.at[pl.ds(0, IDX_PER_G)], rows_v, sem).wait()


def _process_group(rows_v, wt_rows, b_vec, out_v, out_hbm, obase):
    def bag_body(bag, carry):
        rbase = bag * L

        def rbody(i, accs):
            a0, a1, a2, a3 = accs
            r = rbase + i * 8
            a0 = a0 + (rows_v[r, :] + rows_v[r + 4, :])
            a1 = a1 + (rows_v[r + 1, :] + rows_v[r + 5, :])
            a2 = a2 + (rows_v[r + 2, :] + rows_v[r + 6, :])
            a3 = a3 + (rows_v[r + 3, :] + rows_v[r + 7, :])
            return a0, a1, a2, a3

        z = jnp.zeros((D,), jnp.float32)
        a0, a1, a2, a3 = lax.fori_loop(0, L // 8, rbody, (z, z, z, z))
        s = (a0 + a1) + (a2 + a3)

        parts = [b_vec, z, z, z]
        for k in range(D):
            parts[k % 4] = parts[k % 4] + s[k] * wt_rows[k]
        out_v[bag, :] = (parts[0] + parts[1]) + (parts[2] + parts[3])
        return carry

    lax.fori_loop(0, G, bag_body, 0)
    pltpu.sync_copy(out_v, out_hbm.at[pl.ds(obase, G), :])


def _body(text_hbm, table_hbm, wt_hbm, bias_hbm, out_hbm,
          idx0, idx1, rows0, rows1, wt_v, b_v, out_v, sem0, sem1):
    wid = lax.axis_index("s") * NC + lax.axis_index("c")
    tbase = wid * BAGS_PER_W * L
    obase = wid * BAGS_PER_W

    pltpu.sync_copy(wt_hbm, wt_v)
    pltpu.sync_copy(bias_hbm, b_v)
    wt_rows = [wt_v[k, :] for k in range(D)]
    b_vec = b_v[:]

    _fire_group(text_hbm, table_hbm, idx0, rows0, sem0, tbase)

    def outer(g2, carry):
        gA = g2 * 2
        gB = gA + 1
        _fire_group(text_hbm, table_hbm, idx1, rows1, sem1,
                    tbase + gB * IDX_PER_G)
        _drain_group(table_hbm, rows0, sem0)
        _process_group(rows0, wt_rows, b_vec, out_v, out_hbm, obase + gA * G)

        @pl.when(g2 < NG // 2 - 1)
        def _():
            _fire_group(text_hbm, table_hbm, idx0, rows0, sem0,
                        tbase + (gA + 2) * IDX_PER_G)

        _drain_group(table_hbm, rows1, sem1)
        _process_group(rows1, wt_rows, b_vec, out_v, out_hbm, obase + gB * G)
        return carry

    lax.fori_loop(0, NG // 2, outer, 0)


TC_C = 7936
TC_GRID = -(-1000000 // TC_C)


def _tc_transpose_body(xt_ref, out_ref):
    x = xt_ref[...]
    xt = x.T.reshape(TC_C // 8, 8, D)
    pieces = [xt[:, q, :] for q in range(8)]
    out_ref[...] = jnp.concatenate(pieces, axis=1)


def _tc_transpose(table_t):
    out = pl.pallas_call(
        _tc_transpose_body,
        grid=(TC_GRID,),
        in_specs=[pl.BlockSpec((16, TC_C), lambda i: (0, i))],
        out_specs=pl.BlockSpec((TC_C // 8, 128), lambda i: (i, 0)),
        out_shape=jax.ShapeDtypeStruct((125000, 128), jnp.float32),
    )(table_t)
    return out.reshape(1000000, D)


@jax.jit
def kernel(text, table, W, b):
    text_flat = text.reshape(-1).astype(jnp.int32)
    table_lin = _tc_transpose(table.T)
    wt = (W.T / jnp.float32(L)).astype(jnp.float32)
    run = pl.kernel(
        _body,
        out_type=jax.ShapeDtypeStruct((B, D), jnp.float32),
        mesh=plsc.VectorSubcoreMesh(core_axis_name="c", subcore_axis_name="s"),
        compiler_params=pltpu.CompilerParams(use_tc_tiling_on_sc=False),
        scratch_types=[
            pltpu.VMEM((IDX_PER_G,), jnp.int32),
            pltpu.VMEM((IDX_PER_G,), jnp.int32),
            pltpu.VMEM((IDX_PER_G, D), jnp.float32),
            pltpu.VMEM((IDX_PER_G, D), jnp.float32),
            pltpu.VMEM((D, D), jnp.float32),
            pltpu.VMEM((D,), jnp.float32),
            pltpu.VMEM((G, D), jnp.float32),
            pltpu.SemaphoreType.DMA,
            pltpu.SemaphoreType.DMA,
        ],
    )
    return run(text_flat, table_lin, wt, b.astype(jnp.float32))
